# baseline (speedup 1.0000x reference)
import functools

import jax
import jax.numpy as jnp
from jax import lax
from jax.experimental import pallas as pl
from jax.experimental.pallas import tpu as pltpu
from jax.experimental.pallas import tpu_sc as plsc

N = 10000
E = 320000
D = 128
H = 128
NL = 3

NC = 2
NS = 16
NW = NC * NS
CHUNK = 128
CPP = 160
E_PAD = NS * CPP * CHUNK
FAST_CID = 1
A_CHUNKS = 104
B_CHUNKS = CPP - A_CHUNKS
DEG_CHUNKS = CPP // 2
SUB = 40
NSUP = 2
NPAD = 10240
ROWS_PER_TILE = NPAD // NS

@functools.cache
def _mesh():
    return plsc.VectorSubcoreMesh(core_axis_name="c", subcore_axis_name="s",
                                  num_cores=NC, num_subcores=NS)


def _deg_body(dst_hbm, zeros_hbm, ones_hbm, out_hbm, dst_v, ones_v,
              sem0, sem1, shared_deg):
    cid = lax.axis_index("c")
    sid = lax.axis_index("s")
    sems = (sem0, sem1)
    pltpu.sync_copy(ones_hbm, ones_v)
    pltpu.sync_copy(zeros_hbm,
                    shared_deg.at[pl.ds(sid * ROWS_PER_TILE, ROWS_PER_TILE)])
    plsc.subcore_barrier()

    pltpu.sync_copy(dst_hbm.at[sid].at[pl.ds(cid * DEG_CHUNKS, DEG_CHUNKS)],
                    dst_v)

    for b in range(2):
        pltpu.async_copy(ones_v, shared_deg.at[dst_v.at[b]], sems[b],
                         add=True)

    @pl.loop(2, DEG_CHUNKS, step=2)
    def _(c):
        for b in range(2):
            cc = c + b
            pltpu.make_async_copy(ones_v, shared_deg.at[dst_v.at[cc - 2]],
                                  sems[b]).wait()
            pltpu.async_copy(ones_v, shared_deg.at[dst_v.at[cc]], sems[b],
                             add=True)

    for b in range(2):
        pltpu.make_async_copy(ones_v,
                              shared_deg.at[dst_v.at[DEG_CHUNKS - 2 + b]],
                              sems[b]).wait()

    plsc.subcore_barrier()
    pltpu.sync_copy(
        shared_deg.at[pl.ds(sid * ROWS_PER_TILE, ROWS_PER_TILE)],
        out_hbm.at[cid].at[pl.ds(sid * ROWS_PER_TILE, ROWS_PER_TILE)])


@functools.cache
def _deg_kernel():
    return pl.kernel(
        _deg_body,
        out_type=jax.ShapeDtypeStruct((NC, NPAD, H), jnp.float32),
        mesh=_mesh(),
        scratch_types=[
            pltpu.VMEM((DEG_CHUNKS, CHUNK), jnp.int32),
            pltpu.VMEM((CHUNK, H), jnp.float32),
            pltpu.SemaphoreType.DMA,
            pltpu.SemaphoreType.DMA,
            pltpu.VMEM_SHARED((NPAD, H), jnp.float32),
        ],
    )


def _agg_body(h_hbm, src_hbm, dst_hbm, zeros_hbm, out_hbm, src_v, dst_v,
              rows0_v, rows1_v, sem0, sem1, gsem, shared_agg):
    cid = lax.axis_index("c")
    sid = lax.axis_index("s")
    pltpu.sync_copy(zeros_hbm,
                    shared_agg.at[pl.ds(sid * ROWS_PER_TILE, ROWS_PER_TILE)])
    plsc.subcore_barrier()

    rows = (rows0_v, rows1_v)
    sems = (sem0, sem1)
    half = CPP // 2

    @pl.loop(0, NSUP)
    def _(sup):
        base = cid * half + sup * SUB
        pltpu.sync_copy(src_hbm.at[sid].at[pl.ds(base, SUB)], src_v)
        pltpu.sync_copy(dst_hbm.at[sid].at[pl.ds(base, SUB)], dst_v)
        for b in range(2):
            pltpu.async_copy(h_hbm.at[src_v.at[b]], rows[b], gsem).wait()
            pltpu.async_copy(rows[b], shared_agg.at[dst_v.at[b]], sems[b],
                             add=True)

        @pl.loop(2, SUB, step=2)
        def _(c):
            for b in range(2):
                cc = c + b
                pltpu.make_async_copy(rows[b],
                                      shared_agg.at[dst_v.at[cc - 2]],
                                      sems[b]).wait()
                pltpu.async_copy(h_hbm.at[src_v.at[cc]], rows[b],
                                 gsem).wait()
                pltpu.async_copy(rows[b], shared_agg.at[dst_v.at[cc]],
                                 sems[b], add=True)

        for b in range(2):
            pltpu.make_async_copy(rows[b],
                                  shared_agg.at[dst_v.at[SUB - 2 + b]],
                                  sems[b]).wait()

    plsc.subcore_barrier()
    pltpu.sync_copy(
        shared_agg.at[pl.ds(sid * ROWS_PER_TILE, ROWS_PER_TILE)],
        out_hbm.at[cid].at[pl.ds(sid * ROWS_PER_TILE, ROWS_PER_TILE)])


@functools.cache
def _agg_kernel():
    return pl.kernel(
        _agg_body,
        out_type=jax.ShapeDtypeStruct((NC, NPAD, H), jnp.float32),
        mesh=_mesh(),
        scratch_types=[
            pltpu.VMEM((SUB, CHUNK), jnp.int32),
            pltpu.VMEM((SUB, CHUNK), jnp.int32),
            pltpu.VMEM((CHUNK, H), jnp.float32),
            pltpu.VMEM((CHUNK, H), jnp.float32),
            pltpu.SemaphoreType.DMA,
            pltpu.SemaphoreType.DMA,
            pltpu.SemaphoreType.DMA,
            pltpu.VMEM_SHARED((NPAD, H), jnp.float32),
        ],
    )


def _bn(h, gamma, beta, eps):
    mu = jnp.mean(h, axis=0, keepdims=True)
    var = jnp.mean((h - mu) * (h - mu), axis=0, keepdims=True)
    return (h - mu) * lax.rsqrt(var + eps) * gamma + beta


def _dot(a, b):
    return jnp.dot(a, b, preferred_element_type=jnp.float32)


def _dense01_body(h_ref, p0_ref, p1_ref, d0_ref, d1_ref, ws_ref, wn_ref,
                  b_ref, g_ref, be_ref, out_ref):
    deg = jnp.maximum(d0_ref[...] + d1_ref[...], 1.0)
    agg = (p0_ref[...] + p1_ref[...]) / deg
    z = _dot(h_ref[...], ws_ref[...]) + _dot(agg, wn_ref[...]) + b_ref[...]
    r = jnp.maximum(z, 0.0)
    hc = r - jnp.mean(r, axis=0, keepdims=True)
    rms = jnp.sqrt(1e-5 + jnp.sum(hc * hc) / N)
    hp = hc / rms
    out_ref[...] = _bn(hp, g_ref[...], be_ref[...], 1e-5)


_dense01 = pl.pallas_call(
    _dense01_body,
    out_shape=jax.ShapeDtypeStruct((N, H), jnp.float32),
)


def _final_body(h_ref, p0_ref, p1_ref, d0_ref, d1_ref, ws_ref, wn_ref, b_ref,
                g2_ref, be2_ref, fw1_ref, fb1_ref, fg1_ref, fbe1_ref,
                fw2_ref, fb2_ref, fg2_ref, fbe2_ref, lw_ref, lb_ref, out_ref):
    deg = jnp.maximum(d0_ref[...] + d1_ref[...], 1.0)
    agg = (p0_ref[...] + p1_ref[...]) / deg
    z = _dot(h_ref[...], ws_ref[...]) + _dot(agg, wn_ref[...]) + b_ref[...]
    h2 = _bn(z, g2_ref[...], be2_ref[...], 1e-5)
    t = _dot(h2, fw1_ref[...]) + fb1_ref[...]
    t = jnp.maximum(_bn(t, fg1_ref[...], fbe1_ref[...], 1e-3), 0.0)
    t = _dot(t, fw2_ref[...]) + fb2_ref[...]
    t = jnp.maximum(_bn(t, fg2_ref[...], fbe2_ref[...], 1e-3), 0.0)
    out_ref[...] = _dot(t, lw_ref[...]) + lb_ref[...]


_final = pl.pallas_call(
    _final_body,
    out_shape=jax.ShapeDtypeStruct((N, H), jnp.float32),
)


def kernel(x, edge_index, params):
    src = edge_index[0].astype(jnp.int32)
    dst = edge_index[1].astype(jnp.int32)
    pad = E_PAD - E
    pad_dst = N + jnp.arange(pad, dtype=jnp.int32) % (NPAD - N)
    pad_src = jnp.arange(pad, dtype=jnp.int32) % N
    src_p = jnp.concatenate([src, pad_src]).reshape(NS, CPP, CHUNK)
    dst_p = jnp.concatenate([dst, pad_dst]).reshape(NS, CPP, CHUNK)
    zeros_blk = jnp.zeros((ROWS_PER_TILE, H), jnp.float32)
    ones_blk = jnp.ones((CHUNK, H), jnp.float32)

    degp = _deg_kernel()(dst_p, zeros_blk, ones_blk)
    d0 = degp[0, :N, 0:1]
    d1 = degp[1, :N, 0:1]

    def row(v):
        return v.reshape(1, H)

    h = x
    for i in range(NL):
        p = params['sage'][i]
        bn = params['bn'][i]
        aggp = _agg_kernel()(h, src_p, dst_p, zeros_blk)
        p0 = aggp[0, :N]
        p1 = aggp[1, :N]
        if i < NL - 1:
            h = _dense01(h, p0, p1, d0, d1, p['W_self'], p['W_neigh'],
                         row(p['b']), row(bn['gamma']), row(bn['beta']))
        else:
            fc1, fc2 = params['fc']
            lat = params['latent']
            h = _final(h, p0, p1, d0, d1, p['W_self'], p['W_neigh'],
                       row(p['b']), row(bn['gamma']), row(bn['beta']),
                       fc1['W'], row(fc1['b']), row(fc1['gamma']),
                       row(fc1['beta']),
                       fc2['W'], row(fc2['b']), row(fc2['gamma']),
                       row(fc2['beta']),
                       lat['W'], row(lat['b']))
    return h


# speedup vs baseline: 1.0293x; 1.0293x over previous
import functools

import jax
import jax.numpy as jnp
from jax import lax
from jax.experimental import pallas as pl
from jax.experimental.pallas import tpu as pltpu
from jax.experimental.pallas import tpu_sc as plsc

N = 10000
E = 320000
D = 128
H = 128
NL = 3

NC = 2
NS = 16
NW = NC * NS
CHUNK = 128
CPP = 160
E_PAD = NS * CPP * CHUNK
FAST_CID = 1
A_CHUNKS = 104
B_CHUNKS = CPP - A_CHUNKS
DEG_CHUNKS = CPP // 2
SUB = 40
NSUP = 2
NPAD = 10240
ROWS_PER_TILE = NPAD // NS

@functools.cache
def _mesh():
    return plsc.VectorSubcoreMesh(core_axis_name="c", subcore_axis_name="s",
                                  num_cores=NC, num_subcores=NS)


def _deg_body(dst_hbm, zeros_hbm, ones_hbm, out_hbm, dst_v, ones_v,
              sem0, sem1, shared_deg):
    cid = lax.axis_index("c")
    sid = lax.axis_index("s")
    sems = (sem0, sem1)
    pltpu.sync_copy(ones_hbm, ones_v)
    pltpu.sync_copy(zeros_hbm,
                    shared_deg.at[pl.ds(sid * ROWS_PER_TILE, ROWS_PER_TILE)])
    plsc.subcore_barrier()

    pltpu.sync_copy(dst_hbm.at[sid].at[pl.ds(cid * DEG_CHUNKS, DEG_CHUNKS)],
                    dst_v)

    for b in range(2):
        pltpu.async_copy(ones_v, shared_deg.at[dst_v.at[b]], sems[b],
                         add=True)

    @pl.loop(2, DEG_CHUNKS, step=2)
    def _(c):
        for b in range(2):
            cc = c + b
            pltpu.make_async_copy(ones_v, shared_deg.at[dst_v.at[cc - 2]],
                                  sems[b]).wait()
            pltpu.async_copy(ones_v, shared_deg.at[dst_v.at[cc]], sems[b],
                             add=True)

    for b in range(2):
        pltpu.make_async_copy(ones_v,
                              shared_deg.at[dst_v.at[DEG_CHUNKS - 2 + b]],
                              sems[b]).wait()

    plsc.subcore_barrier()
    pltpu.sync_copy(
        shared_deg.at[pl.ds(sid * ROWS_PER_TILE, ROWS_PER_TILE)],
        out_hbm.at[cid].at[pl.ds(sid * ROWS_PER_TILE, ROWS_PER_TILE)])


@functools.cache
def _deg_kernel():
    return pl.kernel(
        _deg_body,
        out_type=jax.ShapeDtypeStruct((NC, NPAD, H), jnp.float32),
        mesh=_mesh(),
        scratch_types=[
            pltpu.VMEM((DEG_CHUNKS, CHUNK), jnp.int32),
            pltpu.VMEM((CHUNK, H), jnp.float32),
            pltpu.SemaphoreType.DMA,
            pltpu.SemaphoreType.DMA,
            pltpu.VMEM_SHARED((NPAD, H), jnp.float32),
        ],
    )


def _agg_body(h_hbm, src_hbm, dst_hbm, zeros_hbm, out_hbm, src_v, dst_v,
              rows0_v, rows1_v, sem0, sem1, gsem, shared_agg):
    cid = lax.axis_index("c")
    sid = lax.axis_index("s")
    pltpu.sync_copy(zeros_hbm,
                    shared_agg.at[pl.ds(sid * ROWS_PER_TILE, ROWS_PER_TILE)])
    plsc.subcore_barrier()

    rows = (rows0_v, rows1_v)
    sems = (sem0, sem1)
    half = CPP // 2

    @pl.loop(0, NSUP)
    def _(sup):
        base = cid * half + sup * SUB
        pltpu.sync_copy(src_hbm.at[sid].at[pl.ds(base, SUB)], src_v)
        pltpu.sync_copy(dst_hbm.at[sid].at[pl.ds(base, SUB)], dst_v)
        for b in range(2):
            pltpu.async_copy(h_hbm.at[src_v.at[b]], rows[b], gsem).wait()
            pltpu.async_copy(rows[b], shared_agg.at[dst_v.at[b]], sems[b],
                             add=True)

        @pl.loop(2, SUB, step=2)
        def _(c):
            for b in range(2):
                cc = c + b
                pltpu.make_async_copy(rows[b],
                                      shared_agg.at[dst_v.at[cc - 2]],
                                      sems[b]).wait()
                pltpu.async_copy(h_hbm.at[src_v.at[cc]], rows[b],
                                 gsem).wait()
                pltpu.async_copy(rows[b], shared_agg.at[dst_v.at[cc]],
                                 sems[b], add=True)

        for b in range(2):
            pltpu.make_async_copy(rows[b],
                                  shared_agg.at[dst_v.at[SUB - 2 + b]],
                                  sems[b]).wait()

    plsc.subcore_barrier()
    pltpu.sync_copy(
        shared_agg.at[pl.ds(sid * ROWS_PER_TILE, ROWS_PER_TILE)],
        out_hbm.at[cid].at[pl.ds(sid * ROWS_PER_TILE, ROWS_PER_TILE)])


@functools.cache
def _agg_kernel():
    return pl.kernel(
        _agg_body,
        out_type=jax.ShapeDtypeStruct((NC, NPAD, H), jnp.float32),
        mesh=_mesh(),
        scratch_types=[
            pltpu.VMEM((SUB, CHUNK), jnp.int32),
            pltpu.VMEM((SUB, CHUNK), jnp.int32),
            pltpu.VMEM((CHUNK, H), jnp.float32),
            pltpu.VMEM((CHUNK, H), jnp.float32),
            pltpu.SemaphoreType.DMA,
            pltpu.SemaphoreType.DMA,
            pltpu.SemaphoreType.DMA,
            pltpu.VMEM_SHARED((NPAD, H), jnp.float32),
        ],
    )


def _bn(h, gamma, beta, eps):
    mu = jnp.mean(h, axis=0, keepdims=True)
    var = jnp.mean((h - mu) * (h - mu), axis=0, keepdims=True)
    return (h - mu) * lax.rsqrt(var + eps) * gamma + beta


def _dot(a, b):
    return jnp.dot(a, b, preferred_element_type=jnp.float32)


def _dense01_body(h_ref, pp_ref, d0_ref, d1_ref, ws_ref, wn_ref,
                  b_ref, g_ref, be_ref, out_ref):
    deg = jnp.maximum(d0_ref[...] + d1_ref[...], 1.0)
    agg = (pp_ref[0, :N, :] + pp_ref[1, :N, :]) / deg
    z = _dot(h_ref[...], ws_ref[...]) + _dot(agg, wn_ref[...]) + b_ref[...]
    r = jnp.maximum(z, 0.0)
    hc = r - jnp.mean(r, axis=0, keepdims=True)
    rms = jnp.sqrt(1e-5 + jnp.sum(hc * hc) / N)
    hp = hc / rms
    out_ref[...] = _bn(hp, g_ref[...], be_ref[...], 1e-5)


_dense01 = pl.pallas_call(
    _dense01_body,
    out_shape=jax.ShapeDtypeStruct((N, H), jnp.float32),
)


def _final_body(h_ref, pp_ref, d0_ref, d1_ref, ws_ref, wn_ref, b_ref,
                g2_ref, be2_ref, fw1_ref, fb1_ref, fg1_ref, fbe1_ref,
                fw2_ref, fb2_ref, fg2_ref, fbe2_ref, lw_ref, lb_ref, out_ref):
    deg = jnp.maximum(d0_ref[...] + d1_ref[...], 1.0)
    agg = (pp_ref[0, :N, :] + pp_ref[1, :N, :]) / deg
    z = _dot(h_ref[...], ws_ref[...]) + _dot(agg, wn_ref[...]) + b_ref[...]
    h2 = _bn(z, g2_ref[...], be2_ref[...], 1e-5)
    t = _dot(h2, fw1_ref[...]) + fb1_ref[...]
    t = jnp.maximum(_bn(t, fg1_ref[...], fbe1_ref[...], 1e-3), 0.0)
    t = _dot(t, fw2_ref[...]) + fb2_ref[...]
    t = jnp.maximum(_bn(t, fg2_ref[...], fbe2_ref[...], 1e-3), 0.0)
    out_ref[...] = _dot(t, lw_ref[...]) + lb_ref[...]


_final = pl.pallas_call(
    _final_body,
    out_shape=jax.ShapeDtypeStruct((N, H), jnp.float32),
)


def kernel(x, edge_index, params):
    src = edge_index[0].astype(jnp.int32)
    dst = edge_index[1].astype(jnp.int32)
    pad = E_PAD - E
    pad_dst = N + jnp.arange(pad, dtype=jnp.int32) % (NPAD - N)
    pad_src = jnp.arange(pad, dtype=jnp.int32) % N
    src_p = jnp.concatenate([src, pad_src]).reshape(NS, CPP, CHUNK)
    dst_p = jnp.concatenate([dst, pad_dst]).reshape(NS, CPP, CHUNK)
    zeros_blk = jnp.zeros((ROWS_PER_TILE, H), jnp.float32)
    ones_blk = jnp.ones((CHUNK, H), jnp.float32)

    degp = _deg_kernel()(dst_p, zeros_blk, ones_blk)
    d0 = degp[0, :N, 0:1]
    d1 = degp[1, :N, 0:1]

    def row(v):
        return v.reshape(1, H)

    h = x
    for i in range(NL):
        p = params['sage'][i]
        bn = params['bn'][i]
        aggp = _agg_kernel()(h, src_p, dst_p, zeros_blk)
        if i < NL - 1:
            h = _dense01(h, aggp, d0, d1, p['W_self'], p['W_neigh'],
                         row(p['b']), row(bn['gamma']), row(bn['beta']))
        else:
            fc1, fc2 = params['fc']
            lat = params['latent']
            h = _final(h, aggp, d0, d1, p['W_self'], p['W_neigh'],
                       row(p['b']), row(bn['gamma']), row(bn['beta']),
                       fc1['W'], row(fc1['b']), row(fc1['gamma']),
                       row(fc1['beta']),
                       fc2['W'], row(fc2['b']), row(fc2['gamma']),
                       row(fc2['beta']),
                       lat['W'], row(lat['b']))
    return h


# speedup vs baseline: 1.0326x; 1.0032x over previous
import functools

import jax
import jax.numpy as jnp
from jax import lax
from jax.experimental import pallas as pl
from jax.experimental.pallas import tpu as pltpu
from jax.experimental.pallas import tpu_sc as plsc

N = 10000
E = 320000
D = 128
H = 128
NL = 3

NC = 2
NS = 16
NW = NC * NS
CHUNK = 128
CPP = 160
E_PAD = NS * CPP * CHUNK
DEG_CHUNKS = CPP // 2
SUB = 40
NSUP = 2
NPAD = 10240
ROWS_PER_TILE = NPAD // NS

@functools.cache
def _mesh():
    return plsc.VectorSubcoreMesh(core_axis_name="c", subcore_axis_name="s",
                                  num_cores=NC, num_subcores=NS)


def _deg_body(dst_hbm, zeros_hbm, ones_hbm, out_hbm, dst_v, ones_v,
              sem0, sem1, shared_deg):
    cid = lax.axis_index("c")
    sid = lax.axis_index("s")
    sems = (sem0, sem1)
    pltpu.sync_copy(ones_hbm, ones_v)
    pltpu.sync_copy(zeros_hbm,
                    shared_deg.at[pl.ds(sid * ROWS_PER_TILE, ROWS_PER_TILE)])
    plsc.subcore_barrier()

    pltpu.sync_copy(dst_hbm.at[sid].at[pl.ds(cid * DEG_CHUNKS, DEG_CHUNKS)],
                    dst_v)

    for b in range(2):
        pltpu.async_copy(ones_v, shared_deg.at[dst_v.at[b]], sems[b],
                         add=True)

    @pl.loop(2, DEG_CHUNKS, step=2)
    def _(c):
        for b in range(2):
            cc = c + b
            pltpu.make_async_copy(ones_v, shared_deg.at[dst_v.at[cc - 2]],
                                  sems[b]).wait()
            pltpu.async_copy(ones_v, shared_deg.at[dst_v.at[cc]], sems[b],
                             add=True)

    for b in range(2):
        pltpu.make_async_copy(ones_v,
                              shared_deg.at[dst_v.at[DEG_CHUNKS - 2 + b]],
                              sems[b]).wait()

    plsc.subcore_barrier()
    pltpu.sync_copy(
        shared_deg.at[pl.ds(sid * ROWS_PER_TILE, ROWS_PER_TILE)],
        out_hbm.at[cid].at[pl.ds(sid * ROWS_PER_TILE, ROWS_PER_TILE)])


@functools.cache
def _deg_kernel():
    return pl.kernel(
        _deg_body,
        out_type=jax.ShapeDtypeStruct((NC, NPAD, H), jnp.float32),
        mesh=_mesh(),
        scratch_types=[
            pltpu.VMEM((DEG_CHUNKS, CHUNK), jnp.int32),
            pltpu.VMEM((CHUNK, H), jnp.float32),
            pltpu.SemaphoreType.DMA,
            pltpu.SemaphoreType.DMA,
            pltpu.VMEM_SHARED((NPAD, H), jnp.float32),
        ],
    )


def _agg_body(h_hbm, src_hbm, dst_hbm, zeros_hbm, out_hbm, src_v, dst_v,
              rows0_v, rows1_v, sem0, sem1, gsem, shared_agg):
    cid = lax.axis_index("c")
    sid = lax.axis_index("s")
    pltpu.sync_copy(zeros_hbm,
                    shared_agg.at[pl.ds(sid * ROWS_PER_TILE, ROWS_PER_TILE)])
    plsc.subcore_barrier()

    rows = (rows0_v, rows1_v)
    sems = (sem0, sem1)
    half = CPP // 2

    @pl.loop(0, NSUP)
    def _(sup):
        base = cid * half + sup * SUB
        pltpu.sync_copy(src_hbm.at[sid].at[pl.ds(base, SUB)], src_v)
        pltpu.sync_copy(dst_hbm.at[sid].at[pl.ds(base, SUB)], dst_v)
        for b in range(2):
            pltpu.async_copy(h_hbm.at[src_v.at[b]], rows[b], gsem).wait()
            pltpu.async_copy(rows[b], shared_agg.at[dst_v.at[b]], sems[b],
                             add=True)

        @pl.loop(2, SUB, step=2)
        def _(c):
            for b in range(2):
                cc = c + b
                pltpu.make_async_copy(rows[b],
                                      shared_agg.at[dst_v.at[cc - 2]],
                                      sems[b]).wait()
                pltpu.async_copy(h_hbm.at[src_v.at[cc]], rows[b],
                                 gsem).wait()
                pltpu.async_copy(rows[b], shared_agg.at[dst_v.at[cc]],
                                 sems[b], add=True)

        for b in range(2):
            pltpu.make_async_copy(rows[b],
                                  shared_agg.at[dst_v.at[SUB - 2 + b]],
                                  sems[b]).wait()

    plsc.subcore_barrier()
    pltpu.sync_copy(
        shared_agg.at[pl.ds(sid * ROWS_PER_TILE, ROWS_PER_TILE)],
        out_hbm.at[cid].at[pl.ds(sid * ROWS_PER_TILE, ROWS_PER_TILE)])


@functools.cache
def _agg_kernel():
    return pl.kernel(
        _agg_body,
        out_type=jax.ShapeDtypeStruct((NC, NPAD, H), jnp.float32),
        mesh=_mesh(),
        scratch_types=[
            pltpu.VMEM((SUB, CHUNK), jnp.int32),
            pltpu.VMEM((SUB, CHUNK), jnp.int32),
            pltpu.VMEM((CHUNK, H), jnp.float32),
            pltpu.VMEM((CHUNK, H), jnp.float32),
            pltpu.SemaphoreType.DMA,
            pltpu.SemaphoreType.DMA,
            pltpu.SemaphoreType.DMA,
            pltpu.VMEM_SHARED((NPAD, H), jnp.float32),
        ],
    )


def _bn(h, gamma, beta, eps):
    mu = jnp.mean(h, axis=0, keepdims=True)
    var = jnp.mean((h - mu) * (h - mu), axis=0, keepdims=True)
    return (h - mu) * lax.rsqrt(var + eps) * gamma + beta


def _dot(a, b):
    return jnp.dot(a, b, preferred_element_type=jnp.float32)


def _dense01_body(h_ref, pp_ref, d0_ref, d1_ref, ws_ref, wn_ref,
                  b_ref, g_ref, be_ref, out_ref):
    deg = jnp.maximum(d0_ref[...] + d1_ref[...], 1.0)
    agg = (pp_ref[0, :N, :] + pp_ref[1, :N, :]) / deg
    z = _dot(h_ref[...], ws_ref[...]) + _dot(agg, wn_ref[...]) + b_ref[...]
    r = jnp.maximum(z, 0.0)
    hc = r - jnp.mean(r, axis=0, keepdims=True)
    rms = jnp.sqrt(1e-5 + jnp.sum(hc * hc) / N)
    hp = hc / rms
    out_ref[...] = _bn(hp, g_ref[...], be_ref[...], 1e-5)


_dense01 = pl.pallas_call(
    _dense01_body,
    out_shape=jax.ShapeDtypeStruct((N, H), jnp.float32),
)


def _final_body(h_ref, pp_ref, d0_ref, d1_ref, ws_ref, wn_ref, b_ref,
                g2_ref, be2_ref, fw1_ref, fb1_ref, fg1_ref, fbe1_ref,
                fw2_ref, fb2_ref, fg2_ref, fbe2_ref, lw_ref, lb_ref, out_ref):
    deg = jnp.maximum(d0_ref[...] + d1_ref[...], 1.0)
    agg = (pp_ref[0, :N, :] + pp_ref[1, :N, :]) / deg
    z = _dot(h_ref[...], ws_ref[...]) + _dot(agg, wn_ref[...]) + b_ref[...]
    h2 = _bn(z, g2_ref[...], be2_ref[...], 1e-5)
    t = _dot(h2, fw1_ref[...]) + fb1_ref[...]
    t = jnp.maximum(_bn(t, fg1_ref[...], fbe1_ref[...], 1e-3), 0.0)
    t = _dot(t, fw2_ref[...]) + fb2_ref[...]
    t = jnp.maximum(_bn(t, fg2_ref[...], fbe2_ref[...], 1e-3), 0.0)
    out_ref[...] = _dot(t, lw_ref[...]) + lb_ref[...]


_final = pl.pallas_call(
    _final_body,
    out_shape=jax.ShapeDtypeStruct((N, H), jnp.float32),
)


def kernel(x, edge_index, params):
    src = edge_index[0].astype(jnp.int32)
    dst = edge_index[1].astype(jnp.int32)
    pad = E_PAD - E
    pad_dst = N + jnp.arange(pad, dtype=jnp.int32) % (NPAD - N)
    pad_src = jnp.arange(pad, dtype=jnp.int32) % N
    src_p = jnp.concatenate([src, pad_src]).reshape(NS, CPP, CHUNK)
    dst_p = jnp.concatenate([dst, pad_dst]).reshape(NS, CPP, CHUNK)
    zeros_blk = jnp.zeros((ROWS_PER_TILE, H), jnp.float32)
    ones_blk = jnp.ones((CHUNK, H), jnp.float32)

    degp = _deg_kernel()(dst_p, zeros_blk, ones_blk)
    d0 = degp[0, :N, 0:1]
    d1 = degp[1, :N, 0:1]

    def row(v):
        return v.reshape(1, H)

    h = x
    for i in range(NL):
        p = params['sage'][i]
        bn = params['bn'][i]
        aggp = _agg_kernel()(h, src_p, dst_p, zeros_blk)
        if i < NL - 1:
            h = _dense01(h, aggp, d0, d1, p['W_self'], p['W_neigh'],
                         row(p['b']), row(bn['gamma']), row(bn['beta']))
        else:
            fc1, fc2 = params['fc']
            lat = params['latent']
            h = _final(h, aggp, d0, d1, p['W_self'], p['W_neigh'],
                       row(p['b']), row(bn['gamma']), row(bn['beta']),
                       fc1['W'], row(fc1['b']), row(fc1['gamma']),
                       row(fc1['beta']),
                       fc2['W'], row(fc2['b']), row(fc2['gamma']),
                       row(fc2['beta']),
                       lat['W'], row(lat['b']))
    return h
